# Initial kernel scaffold; baseline (speedup 1.0000x reference)
import jax
import jax.numpy as jnp
from jax.experimental import pallas as pl


def kernel(x, edge_index, gamma, beta, W0, W1, b):
    raise NotImplementedError("write your pallas kernel here")


# speedup vs baseline: 16.4281x; 16.4281x over previous
import functools

import jax
import jax.numpy as jnp
from jax import lax
from jax.experimental import pallas as pl
from jax.experimental.pallas import tpu as pltpu
from jax.experimental.pallas import tpu_sc as plsc

N = 10000
E = 320000
C_IN = 128
C_OUT = 64
T = 4
THRESH = 0.5
TAU = 0.25
EPS = 1e-5

N_PAD = 10240
EB = 128
NB_E = 157
E_PAD = NB_E * EB * 16
NBLK = 1000
NT = float(N * T)

_F32 = jnp.float32


def _stats_body(xt_ref, sums_ref):
    blk = xt_ref[...]
    s1 = jnp.sum(blk, axis=(0, 1)).reshape(1, C_IN)
    s2 = jnp.sum(blk * blk, axis=(0, 1)).reshape(1, C_IN)
    upd = jnp.concatenate([s1, s2], axis=0)

    @pl.when(pl.program_id(0) == 0)
    def _():
        sums_ref[...] = upd

    @pl.when(pl.program_id(0) != 0)
    def _():
        sums_ref[...] = sums_ref[...] + upd


def _stats(xt):
    return pl.pallas_call(
        _stats_body,
        grid=(N // NBLK,),
        in_specs=[pl.BlockSpec((T, NBLK, C_IN), lambda g: (0, g, 0))],
        out_specs=pl.BlockSpec((2, C_IN), lambda g: (0, 0)),
        out_shape=jax.ShapeDtypeStruct((2, C_IN), _F32),
    )(xt)


def _deg_body(dst3_hbm, deg_hbm, dst_v, ones_v, zero_v, deg_sh):
    c = lax.axis_index("c")
    s = lax.axis_index("s")

    def fill(i, carry):
        ones_v[i, :] = jnp.full((16,), 1.0, _F32)
        zero_v[i, :] = jnp.zeros((16,), _F32)
        return carry

    lax.fori_loop(0, EB, fill, 0)

    for k in range(5):
        pltpu.sync_copy(zero_v, deg_sh.at[pl.ds(s * 640 + k * EB, EB)])
    plsc.subcore_barrier()

    pltpu.sync_copy(dst3_hbm.at[s], dst_v)

    def acc(j, carry):
        pltpu.sync_copy(ones_v, deg_sh.at[dst_v.at[j]], add=True)
        return carry

    lax.fori_loop(0, NB_E, acc, 0)
    plsc.subcore_barrier()

    base = (c * 16 + s) * 320
    pltpu.sync_copy(deg_sh.at[pl.ds(base, 320)],
                    deg_hbm.at[pl.ds(base, 320)])


def _degree_inv(dst3):
    mesh = plsc.VectorSubcoreMesh(core_axis_name="c", subcore_axis_name="s")
    k = functools.partial(
        pl.kernel,
        mesh=mesh,
        compiler_params=pltpu.CompilerParams(use_tc_tiling_on_sc=False),
        out_type=jax.ShapeDtypeStruct((N_PAD, 16), _F32),
        scratch_types=[
            pltpu.VMEM((NB_E, EB), jnp.int32),
            pltpu.VMEM((EB, 16), _F32),
            pltpu.VMEM((EB, 16), _F32),
            pltpu.VMEM_SHARED((N_PAD, 16), _F32),
        ],
    )(_deg_body)
    return k(dst3)


def _main_body(xt_ref, sums_ref, deg_ref, gamma_ref, beta_ref, W0_ref,
               W1_ref, b_ref, z0_ref, z1_ref, z2_ref, z3_ref, y0_ref, y1_ref,
               y2_ref, y3_ref, dinv_ref):
    z_refs = (z0_ref, z1_ref, z2_ref, z3_ref)
    y_refs = (y0_ref, y1_ref, y2_ref, y3_ref)
    mean = sums_ref[0:1, :] / NT
    var = sums_ref[1:2, :] / NT - mean * mean
    inv = lax.rsqrt(var + EPS)
    scale = gamma_ref[...] * inv
    bias = beta_ref[...] - mean * scale
    deg = deg_ref[...][:, 0:1]
    dinv = jnp.where(deg > 0.0, lax.rsqrt(jnp.maximum(deg, 1e-30)), 0.0)
    dinv_ref[...] = jnp.broadcast_to(dinv, (NBLK, 16))
    W0 = W0_ref[...]
    W1 = W1_ref[...]
    bb = b_ref[...]

    mem = jnp.zeros((NBLK, C_IN), _F32)
    for t in range(T):
        h = xt_ref[t] * scale + bias
        mem = mem * TAU + h
        sp = (mem > THRESH).astype(_F32)
        mem = mem * (1.0 - sp)
        y_refs[t][...] = jnp.dot(sp, W0, preferred_element_type=_F32) + bb
        z_refs[t][...] = jnp.dot(sp, W1, preferred_element_type=_F32) * dinv


def _main(xt, sums, deg2d, gamma2d, beta2d, W0, W1, b2d):
    return pl.pallas_call(
        _main_body,
        grid=(N // NBLK,),
        in_specs=[
            pl.BlockSpec((T, NBLK, C_IN), lambda g: (0, g, 0)),
            pl.BlockSpec((2, C_IN), lambda g: (0, 0)),
            pl.BlockSpec((NBLK, 16), lambda g: (g, 0)),
            pl.BlockSpec((1, C_IN), lambda g: (0, 0)),
            pl.BlockSpec((1, C_IN), lambda g: (0, 0)),
            pl.BlockSpec((C_IN, C_OUT), lambda g: (0, 0)),
            pl.BlockSpec((C_IN, C_OUT), lambda g: (0, 0)),
            pl.BlockSpec((1, C_OUT), lambda g: (0, 0)),
        ],
        out_specs=(
            [pl.BlockSpec((NBLK, C_OUT), lambda g: (g, 0)) for _ in range(8)]
            + [pl.BlockSpec((NBLK, 16), lambda g: (g, 0))]
        ),
        out_shape=(
            [jax.ShapeDtypeStruct((N_PAD, C_OUT), _F32) for _ in range(4)]
            + [jax.ShapeDtypeStruct((N, C_OUT), _F32) for _ in range(4)]
            + [jax.ShapeDtypeStruct((N, 16), _F32)]
        ),
    )(xt, sums, deg2d, gamma2d, beta2d, W0, W1, b2d)


def _edge_body(src3_hbm, dst3_hbm, z0_hbm, z1_hbm, z2_hbm, z3_hbm, y0_hbm,
               y1_hbm, y2_hbm, y3_hbm, dinv_hbm, o0_hbm, o1_hbm, o2_hbm,
               o3_hbm, src_v, dst_v, rows_v, zeros_v, acc_v, y_v, dinv_v,
               acc_sh, gsem):
    c = lax.axis_index("c")
    s = lax.axis_index("s")

    def zrow(i, carry):
        for q in range(C_OUT // 16):
            zeros_v[i, pl.ds(q * 16, 16)] = jnp.zeros((16,), _F32)
        return carry

    lax.fori_loop(0, EB, zrow, 0)

    def zero_acc():
        for k in range(5):
            pltpu.sync_copy(zeros_v, acc_sh.at[pl.ds(s * 640 + k * EB, EB)])

    zero_acc()
    plsc.subcore_barrier()

    pltpu.sync_copy(src3_hbm.at[s], src_v)
    pltpu.sync_copy(dst3_hbm.at[s], dst_v)

    def run(z_hbm, y_hbm, out_hbm, last):
        def edge(j, carry):
            pltpu.async_copy(z_hbm.at[src_v.at[j]], rows_v, gsem).wait()
            pltpu.sync_copy(rows_v, acc_sh.at[dst_v.at[j]], add=True)
            return carry

        lax.fori_loop(0, NB_E, edge, 0)
        plsc.subcore_barrier()

        def wb_chunk(r0, cn):
            pltpu.sync_copy(acc_sh.at[pl.ds(r0, cn)], acc_v.at[pl.ds(0, cn)])
            pltpu.sync_copy(y_hbm.at[pl.ds(r0, cn)], y_v.at[pl.ds(0, cn)])
            pltpu.sync_copy(dinv_hbm.at[pl.ds(r0, cn)],
                            dinv_v.at[pl.ds(0, cn)])

            def wb(i, carry):
                dv = dinv_v[i, :]
                for q in range(C_OUT // 16):
                    sl = pl.ds(q * 16, 16)
                    acc_v[i, sl] = y_v[i, sl] + acc_v[i, sl] * dv
                return carry

            lax.fori_loop(0, cn, wb, 0)
            pltpu.sync_copy(acc_v.at[pl.ds(0, cn)], out_hbm.at[pl.ds(r0, cn)])

        for k in range(5):
            cid = s + 16 * k

            @pl.when(cid < 78)
            def _():
                wb_chunk(cid * EB, EB)

            @pl.when(cid == 78)
            def _():
                wb_chunk(78 * EB, 16)

        if not last:
            plsc.subcore_barrier()
            zero_acc()
            plsc.subcore_barrier()

    @pl.when(c == 0)
    def _():
        run(z0_hbm, y0_hbm, o0_hbm, False)
        run(z1_hbm, y1_hbm, o1_hbm, True)

    @pl.when(c == 1)
    def _():
        run(z2_hbm, y2_hbm, o2_hbm, False)
        run(z3_hbm, y3_hbm, o3_hbm, True)


def _edge_pass(src3, dst3, zs, ys, dinv):
    mesh = plsc.VectorSubcoreMesh(core_axis_name="c", subcore_axis_name="s")
    k = functools.partial(
        pl.kernel,
        mesh=mesh,
        compiler_params=pltpu.CompilerParams(use_tc_tiling_on_sc=False),
        out_type=[jax.ShapeDtypeStruct((N, C_OUT), _F32) for _ in range(4)],
        scratch_types=[
            pltpu.VMEM((NB_E, EB), jnp.int32),
            pltpu.VMEM((NB_E, EB), jnp.int32),
            pltpu.VMEM((EB, C_OUT), _F32),
            pltpu.VMEM((EB, C_OUT), _F32),
            pltpu.VMEM((EB, C_OUT), _F32),
            pltpu.VMEM((EB, C_OUT), _F32),
            pltpu.VMEM((EB, 16), _F32),
            pltpu.VMEM_SHARED((N_PAD, C_OUT), _F32),
            pltpu.SemaphoreType.DMA,
        ],
    )(_edge_body)
    return k(src3, dst3, *zs, *ys, dinv)


def kernel(x, edge_index, gamma, beta, W0, W1, b):
    xt = jnp.transpose(x, (2, 0, 1))

    src = edge_index[0]
    dst = edge_index[1]
    pad = E_PAD - E
    padidx = N + (jnp.arange(pad, dtype=jnp.int32) % 200)
    src3 = jnp.concatenate([src, padidx]).reshape(16, NB_E, EB)
    dst3 = jnp.concatenate([dst, padidx]).reshape(16, NB_E, EB)

    sums = _stats(xt)
    deg2d = _degree_inv(dst3)

    outs = _main(xt, sums, deg2d,
                 gamma.reshape(1, C_IN), beta.reshape(1, C_IN),
                 W0, W1, b.reshape(1, C_OUT))
    zs, ys, dinv = outs[0:4], outs[4:8], outs[8]

    o0, o1, o2, o3 = _edge_pass(src3, dst3, zs, ys, dinv)
    return jnp.stack([o0, o1, o2, o3], axis=2)


# speedup vs baseline: 19.7725x; 1.2036x over previous
import functools

import jax
import jax.numpy as jnp
from jax import lax
from jax.experimental import pallas as pl
from jax.experimental.pallas import tpu as pltpu
from jax.experimental.pallas import tpu_sc as plsc

N = 10000
E = 320000
C_IN = 128
C_OUT = 64
T = 4
THRESH = 0.5
TAU = 0.25
EPS = 1e-5

N_PAD = 10240
EB = 128
NB_E = 157
E_PAD = NB_E * EB * 16
NBLK = 1000
NT = float(N * T)

_F32 = jnp.float32


def _stats_body(xt_ref, sums_ref):
    blk = xt_ref[...]
    s1 = jnp.sum(blk, axis=(0, 1)).reshape(1, C_IN)
    s2 = jnp.sum(blk * blk, axis=(0, 1)).reshape(1, C_IN)
    upd = jnp.concatenate([s1, s2], axis=0)

    @pl.when(pl.program_id(0) == 0)
    def _():
        sums_ref[...] = upd

    @pl.when(pl.program_id(0) != 0)
    def _():
        sums_ref[...] = sums_ref[...] + upd


def _stats(xt):
    return pl.pallas_call(
        _stats_body,
        grid=(N // NBLK,),
        in_specs=[pl.BlockSpec((T, NBLK, C_IN), lambda g: (0, g, 0))],
        out_specs=pl.BlockSpec((2, C_IN), lambda g: (0, 0)),
        out_shape=jax.ShapeDtypeStruct((2, C_IN), _F32),
    )(xt)


def _deg_body(dst3_hbm, deg_hbm, dst_v, ones_v, zero_v, deg_sh):
    c = lax.axis_index("c")
    s = lax.axis_index("s")

    def fill(i, carry):
        ones_v[i, :] = jnp.full((16,), 1.0, _F32)
        zero_v[i, :] = jnp.zeros((16,), _F32)
        return carry

    lax.fori_loop(0, EB, fill, 0)

    for k in range(5):
        pltpu.sync_copy(zero_v, deg_sh.at[pl.ds(s * 640 + k * EB, EB)])
    plsc.subcore_barrier()

    pltpu.sync_copy(dst3_hbm.at[s], dst_v)

    def acc(j, carry):
        pltpu.sync_copy(ones_v, deg_sh.at[dst_v.at[j]], add=True)
        return carry

    lax.fori_loop(0, NB_E, acc, 0)
    plsc.subcore_barrier()

    base = (c * 16 + s) * 320
    pltpu.sync_copy(deg_sh.at[pl.ds(base, 320)],
                    deg_hbm.at[pl.ds(base, 320)])


def _degree_inv(dst3):
    mesh = plsc.VectorSubcoreMesh(core_axis_name="c", subcore_axis_name="s")
    k = functools.partial(
        pl.kernel,
        mesh=mesh,
        compiler_params=pltpu.CompilerParams(use_tc_tiling_on_sc=False),
        out_type=jax.ShapeDtypeStruct((N_PAD, 16), _F32),
        scratch_types=[
            pltpu.VMEM((NB_E, EB), jnp.int32),
            pltpu.VMEM((EB, 16), _F32),
            pltpu.VMEM((EB, 16), _F32),
            pltpu.VMEM_SHARED((N_PAD, 16), _F32),
        ],
    )(_deg_body)
    return k(dst3)


def _main_body(xt_ref, sums_ref, deg_ref, gamma_ref, beta_ref, W0_ref,
               W1_ref, b_ref, z0_ref, z1_ref, z2_ref, z3_ref, y0_ref, y1_ref,
               y2_ref, y3_ref, dinv_ref):
    z_refs = (z0_ref, z1_ref, z2_ref, z3_ref)
    y_refs = (y0_ref, y1_ref, y2_ref, y3_ref)
    mean = sums_ref[0:1, :] / NT
    var = sums_ref[1:2, :] / NT - mean * mean
    inv = lax.rsqrt(var + EPS)
    scale = gamma_ref[...] * inv
    bias = beta_ref[...] - mean * scale
    deg = deg_ref[...][:, 0:1]
    dinv = jnp.where(deg > 0.0, lax.rsqrt(jnp.maximum(deg, 1e-30)), 0.0)
    dinv_ref[...] = jnp.broadcast_to(dinv, (NBLK, 16))
    W0 = W0_ref[...]
    W1 = W1_ref[...]
    bb = b_ref[...]

    mem = jnp.zeros((NBLK, C_IN), _F32)
    for t in range(T):
        h = xt_ref[t] * scale + bias
        mem = mem * TAU + h
        sp = (mem > THRESH).astype(_F32)
        mem = mem * (1.0 - sp)
        y_refs[t][...] = jnp.dot(sp, W0, preferred_element_type=_F32) + bb
        z_refs[t][...] = jnp.dot(sp, W1, preferred_element_type=_F32) * dinv


def _main(xt, sums, deg2d, gamma2d, beta2d, W0, W1, b2d):
    return pl.pallas_call(
        _main_body,
        grid=(N // NBLK,),
        in_specs=[
            pl.BlockSpec((T, NBLK, C_IN), lambda g: (0, g, 0)),
            pl.BlockSpec((2, C_IN), lambda g: (0, 0)),
            pl.BlockSpec((NBLK, 16), lambda g: (g, 0)),
            pl.BlockSpec((1, C_IN), lambda g: (0, 0)),
            pl.BlockSpec((1, C_IN), lambda g: (0, 0)),
            pl.BlockSpec((C_IN, C_OUT), lambda g: (0, 0)),
            pl.BlockSpec((C_IN, C_OUT), lambda g: (0, 0)),
            pl.BlockSpec((1, C_OUT), lambda g: (0, 0)),
        ],
        out_specs=(
            [pl.BlockSpec((NBLK, C_OUT), lambda g: (g, 0)) for _ in range(8)]
            + [pl.BlockSpec((NBLK, 16), lambda g: (g, 0))]
        ),
        out_shape=(
            [jax.ShapeDtypeStruct((N_PAD, C_OUT), _F32) for _ in range(4)]
            + [jax.ShapeDtypeStruct((N, C_OUT), _F32) for _ in range(4)]
            + [jax.ShapeDtypeStruct((N, 16), _F32)]
        ),
    )(xt, sums, deg2d, gamma2d, beta2d, W0, W1, b2d)


def _edge_body(src3_hbm, dst3_hbm, z0_hbm, z1_hbm, z2_hbm, z3_hbm, y0_hbm,
               y1_hbm, y2_hbm, y3_hbm, dinv_hbm, o0_hbm, o1_hbm, o2_hbm,
               o3_hbm, src_v, dst_v, rows_a, rows_b, zeros_v,
               acc_v, y_v, dinv_v, acc_sh, gsem_a, gsem_b):
    c = lax.axis_index("c")
    s = lax.axis_index("s")
    bufs = (rows_a, rows_b)
    sems = (gsem_a, gsem_b)

    def zrow(i, carry):
        for q in range(C_OUT // 16):
            zeros_v[i, pl.ds(q * 16, 16)] = jnp.zeros((16,), _F32)
        return carry

    lax.fori_loop(0, EB, zrow, 0)

    def zero_acc():
        for k in range(5):
            pltpu.sync_copy(zeros_v, acc_sh.at[pl.ds(s * 640 + k * EB, EB)])

    zero_acc()
    plsc.subcore_barrier()

    pltpu.sync_copy(src3_hbm.at[s], src_v)
    pltpu.sync_copy(dst3_hbm.at[s], dst_v)

    def run(z_hbm, y_hbm, out_hbm, last):
        K = 2

        def group(p, carry):
            j0 = p * K
            hs = [
                pltpu.async_copy(z_hbm.at[src_v.at[j0 + q]], bufs[q],
                                 sems[q])
                for q in range(K)
            ]
            for q in range(K):
                hs[q].wait()
                pltpu.sync_copy(bufs[q], acc_sh.at[dst_v.at[j0 + q]],
                                add=True)
            return carry

        lax.fori_loop(0, (NB_E - 1) // K, group, 0)

        pltpu.async_copy(z_hbm.at[src_v.at[NB_E - 1]], bufs[0],
                         sems[0]).wait()
        pltpu.sync_copy(bufs[0], acc_sh.at[dst_v.at[NB_E - 1]], add=True)
        plsc.subcore_barrier()

        def wb_chunk(r0, cn):
            pltpu.sync_copy(acc_sh.at[pl.ds(r0, cn)], acc_v.at[pl.ds(0, cn)])
            pltpu.sync_copy(y_hbm.at[pl.ds(r0, cn)], y_v.at[pl.ds(0, cn)])
            pltpu.sync_copy(dinv_hbm.at[pl.ds(r0, cn)],
                            dinv_v.at[pl.ds(0, cn)])

            def wb(i, carry):
                dv = dinv_v[i, :]
                for q in range(C_OUT // 16):
                    sl = pl.ds(q * 16, 16)
                    acc_v[i, sl] = y_v[i, sl] + acc_v[i, sl] * dv
                return carry

            lax.fori_loop(0, cn, wb, 0)
            pltpu.sync_copy(acc_v.at[pl.ds(0, cn)], out_hbm.at[pl.ds(r0, cn)])

        for k in range(5):
            cid = s + 16 * k

            @pl.when(cid < 78)
            def _():
                wb_chunk(cid * EB, EB)

            @pl.when(cid == 78)
            def _():
                wb_chunk(78 * EB, 16)

        if not last:
            plsc.subcore_barrier()
            zero_acc()
            plsc.subcore_barrier()

    @pl.when(c == 0)
    def _():
        run(z0_hbm, y0_hbm, o0_hbm, False)
        run(z1_hbm, y1_hbm, o1_hbm, True)

    @pl.when(c == 1)
    def _():
        run(z2_hbm, y2_hbm, o2_hbm, False)
        run(z3_hbm, y3_hbm, o3_hbm, True)


def _edge_pass(src3, dst3, zs, ys, dinv):
    mesh = plsc.VectorSubcoreMesh(core_axis_name="c", subcore_axis_name="s")
    k = functools.partial(
        pl.kernel,
        mesh=mesh,
        compiler_params=pltpu.CompilerParams(use_tc_tiling_on_sc=False),
        out_type=[jax.ShapeDtypeStruct((N, C_OUT), _F32) for _ in range(4)],
        scratch_types=[
            pltpu.VMEM((NB_E, EB), jnp.int32),
            pltpu.VMEM((NB_E, EB), jnp.int32),
            pltpu.VMEM((EB, C_OUT), _F32),
            pltpu.VMEM((EB, C_OUT), _F32),
            pltpu.VMEM((EB, C_OUT), _F32),
            pltpu.VMEM((EB, C_OUT), _F32),
            pltpu.VMEM((EB, C_OUT), _F32),
            pltpu.VMEM((EB, 16), _F32),
            pltpu.VMEM_SHARED((N_PAD, C_OUT), _F32),
            pltpu.SemaphoreType.DMA,
            pltpu.SemaphoreType.DMA,
        ],
    )(_edge_body)
    return k(src3, dst3, *zs, *ys, dinv)


def kernel(x, edge_index, gamma, beta, W0, W1, b):
    xt = jnp.transpose(x, (2, 0, 1))

    src = edge_index[0]
    dst = edge_index[1]
    pad = E_PAD - E
    padidx = N + (jnp.arange(pad, dtype=jnp.int32) % 200)
    src3 = jnp.concatenate([src, padidx]).reshape(16, NB_E, EB)
    dst3 = jnp.concatenate([dst, padidx]).reshape(16, NB_E, EB)

    sums = _stats(xt)
    deg2d = _degree_inv(dst3)

    outs = _main(xt, sums, deg2d,
                 gamma.reshape(1, C_IN), beta.reshape(1, C_IN),
                 W0, W1, b.reshape(1, C_OUT))
    zs, ys, dinv = outs[0:4], outs[4:8], outs[8]

    o0, o1, o2, o3 = _edge_pass(src3, dst3, zs, ys, dinv)
    return jnp.stack([o0, o1, o2, o3], axis=2)


# speedup vs baseline: 20.3898x; 1.0312x over previous
import functools

import jax
import jax.numpy as jnp
from jax import lax
from jax.experimental import pallas as pl
from jax.experimental.pallas import tpu as pltpu
from jax.experimental.pallas import tpu_sc as plsc

N = 10000
E = 320000
C_IN = 128
C_OUT = 64
T = 4
THRESH = 0.5
TAU = 0.25
EPS = 1e-5

N_PAD = 10240
EB = 128
NB_E = 157
E_PAD = NB_E * EB * 16
NBLK = 1000
NT = float(N * T)

_F32 = jnp.float32


def _stats_body(xt_ref, sums_ref):
    blk = xt_ref[...]
    s1 = jnp.sum(blk, axis=(0, 1)).reshape(1, C_IN)
    s2 = jnp.sum(blk * blk, axis=(0, 1)).reshape(1, C_IN)
    upd = jnp.concatenate([s1, s2], axis=0)

    @pl.when(pl.program_id(0) == 0)
    def _():
        sums_ref[...] = upd

    @pl.when(pl.program_id(0) != 0)
    def _():
        sums_ref[...] = sums_ref[...] + upd


def _stats(xt):
    return pl.pallas_call(
        _stats_body,
        grid=(N // NBLK,),
        in_specs=[pl.BlockSpec((T, NBLK, C_IN), lambda g: (0, g, 0))],
        out_specs=pl.BlockSpec((2, C_IN), lambda g: (0, 0)),
        out_shape=jax.ShapeDtypeStruct((2, C_IN), _F32),
    )(xt)


def _deg_body(dst3_hbm, deg_hbm, dst_v, ones_v, zero_v, deg_sh, d0, d1, d2,
              d3):
    dsems = (d0, d1, d2, d3)
    c = lax.axis_index("c")
    s = lax.axis_index("s")

    def fill(i, carry):
        ones_v[i, :] = jnp.full((16,), 1.0, _F32)
        zero_v[i, :] = jnp.zeros((16,), _F32)
        return carry

    lax.fori_loop(0, EB, fill, 0)

    for k in range(5):
        pltpu.sync_copy(zero_v, deg_sh.at[pl.ds(s * 640 + k * EB, EB)])
    plsc.subcore_barrier()

    pltpu.sync_copy(dst3_hbm.at[s], dst_v)

    def acc(p, carry):
        hs = [
            pltpu.async_copy(ones_v, deg_sh.at[dst_v.at[4 * p + q]],
                             dsems[q], add=True)
            for q in range(4)
        ]
        for h in hs:
            h.wait()
        return carry

    lax.fori_loop(0, (NB_E - 1) // 4, acc, 0)
    pltpu.sync_copy(ones_v, deg_sh.at[dst_v.at[NB_E - 1]], add=True)
    plsc.subcore_barrier()

    base = (c * 16 + s) * 320
    pltpu.sync_copy(deg_sh.at[pl.ds(base, 320)],
                    deg_hbm.at[pl.ds(base, 320)])


def _degree_inv(dst3):
    mesh = plsc.VectorSubcoreMesh(core_axis_name="c", subcore_axis_name="s")
    k = functools.partial(
        pl.kernel,
        mesh=mesh,
        compiler_params=pltpu.CompilerParams(use_tc_tiling_on_sc=False),
        out_type=jax.ShapeDtypeStruct((N_PAD, 16), _F32),
        scratch_types=[
            pltpu.VMEM((NB_E, EB), jnp.int32),
            pltpu.VMEM((EB, 16), _F32),
            pltpu.VMEM((EB, 16), _F32),
            pltpu.VMEM_SHARED((N_PAD, 16), _F32),
            pltpu.SemaphoreType.DMA,
            pltpu.SemaphoreType.DMA,
            pltpu.SemaphoreType.DMA,
            pltpu.SemaphoreType.DMA,
        ],
    )(_deg_body)
    return k(dst3)


def _main_body(xt_ref, sums_ref, deg_ref, gamma_ref, beta_ref, W0_ref,
               W1_ref, b_ref, z0_ref, z1_ref, z2_ref, z3_ref, y0_ref, y1_ref,
               y2_ref, y3_ref, dinv_ref):
    z_refs = (z0_ref, z1_ref, z2_ref, z3_ref)
    y_refs = (y0_ref, y1_ref, y2_ref, y3_ref)
    mean = sums_ref[0:1, :] / NT
    var = sums_ref[1:2, :] / NT - mean * mean
    inv = lax.rsqrt(var + EPS)
    scale = gamma_ref[...] * inv
    bias = beta_ref[...] - mean * scale
    deg = deg_ref[...][:, 0:1]
    dinv = jnp.where(deg > 0.0, lax.rsqrt(jnp.maximum(deg, 1e-30)), 0.0)
    dinv_ref[...] = jnp.broadcast_to(dinv, (NBLK, 16))
    W0 = W0_ref[...]
    W1 = W1_ref[...]
    bb = b_ref[...]

    mem = jnp.zeros((NBLK, C_IN), _F32)
    for t in range(T):
        h = xt_ref[t] * scale + bias
        mem = mem * TAU + h
        sp = (mem > THRESH).astype(_F32)
        mem = mem * (1.0 - sp)
        y_refs[t][...] = jnp.dot(sp, W0, preferred_element_type=_F32) + bb
        z_refs[t][...] = jnp.dot(sp, W1, preferred_element_type=_F32) * dinv


def _main(xt, sums, deg2d, gamma2d, beta2d, W0, W1, b2d):
    return pl.pallas_call(
        _main_body,
        grid=(N // NBLK,),
        in_specs=[
            pl.BlockSpec((T, NBLK, C_IN), lambda g: (0, g, 0)),
            pl.BlockSpec((2, C_IN), lambda g: (0, 0)),
            pl.BlockSpec((NBLK, 16), lambda g: (g, 0)),
            pl.BlockSpec((1, C_IN), lambda g: (0, 0)),
            pl.BlockSpec((1, C_IN), lambda g: (0, 0)),
            pl.BlockSpec((C_IN, C_OUT), lambda g: (0, 0)),
            pl.BlockSpec((C_IN, C_OUT), lambda g: (0, 0)),
            pl.BlockSpec((1, C_OUT), lambda g: (0, 0)),
        ],
        out_specs=(
            [pl.BlockSpec((NBLK, C_OUT), lambda g: (g, 0)) for _ in range(8)]
            + [pl.BlockSpec((NBLK, 16), lambda g: (g, 0))]
        ),
        out_shape=(
            [jax.ShapeDtypeStruct((N_PAD, C_OUT), _F32) for _ in range(4)]
            + [jax.ShapeDtypeStruct((N, C_OUT), _F32) for _ in range(4)]
            + [jax.ShapeDtypeStruct((N, 16), _F32)]
        ),
    )(xt, sums, deg2d, gamma2d, beta2d, W0, W1, b2d)


def _edge_body(src3_hbm, dst3_hbm, z0_hbm, z1_hbm, z2_hbm, z3_hbm, y0_hbm,
               y1_hbm, y2_hbm, y3_hbm, dinv_hbm, o0_hbm, o1_hbm, o2_hbm,
               o3_hbm, src_v, dst_v, rows_a, rows_b, zeros_v,
               acc_v, y_v, dinv_v, acc_sh, gsem_a, gsem_b, ssem_a, ssem_b):
    c = lax.axis_index("c")
    s = lax.axis_index("s")
    bufs = (rows_a, rows_b)
    sems = (gsem_a, gsem_b)
    ssems = (ssem_a, ssem_b)

    def zrow(i, carry):
        for q in range(C_OUT // 16):
            zeros_v[i, pl.ds(q * 16, 16)] = jnp.zeros((16,), _F32)
        return carry

    lax.fori_loop(0, EB, zrow, 0)

    def zero_acc():
        for k in range(5):
            pltpu.sync_copy(zeros_v, acc_sh.at[pl.ds(s * 640 + k * EB, EB)])

    zero_acc()
    plsc.subcore_barrier()

    pltpu.sync_copy(src3_hbm.at[s], src_v)
    pltpu.sync_copy(dst3_hbm.at[s], dst_v)

    def run(z_hbm, y_hbm, out_hbm, last):
        K = 2

        def group(p, carry):
            j0 = p * K
            hs = [
                pltpu.async_copy(z_hbm.at[src_v.at[j0 + q]], bufs[q],
                                 sems[q])
                for q in range(K)
            ]
            ss = []
            for q in range(K):
                hs[q].wait()
                ss.append(pltpu.async_copy(
                    bufs[q], acc_sh.at[dst_v.at[j0 + q]], ssems[q],
                    add=True))
            for q in range(K):
                ss[q].wait()
            return carry

        lax.fori_loop(0, (NB_E - 1) // K, group, 0)

        pltpu.async_copy(z_hbm.at[src_v.at[NB_E - 1]], bufs[0],
                         sems[0]).wait()
        pltpu.sync_copy(bufs[0], acc_sh.at[dst_v.at[NB_E - 1]], add=True)
        plsc.subcore_barrier()

        def wb_chunk(r0, cn):
            pltpu.sync_copy(acc_sh.at[pl.ds(r0, cn)], acc_v.at[pl.ds(0, cn)])
            pltpu.sync_copy(y_hbm.at[pl.ds(r0, cn)], y_v.at[pl.ds(0, cn)])
            pltpu.sync_copy(dinv_hbm.at[pl.ds(r0, cn)],
                            dinv_v.at[pl.ds(0, cn)])

            def wb(i, carry):
                dv = dinv_v[i, :]
                for q in range(C_OUT // 16):
                    sl = pl.ds(q * 16, 16)
                    acc_v[i, sl] = y_v[i, sl] + acc_v[i, sl] * dv
                return carry

            lax.fori_loop(0, cn, wb, 0)
            pltpu.sync_copy(acc_v.at[pl.ds(0, cn)], out_hbm.at[pl.ds(r0, cn)])

        for k in range(5):
            cid = s + 16 * k

            @pl.when(cid < 78)
            def _():
                wb_chunk(cid * EB, EB)

            @pl.when(cid == 78)
            def _():
                wb_chunk(78 * EB, 16)

        if not last:
            plsc.subcore_barrier()
            zero_acc()
            plsc.subcore_barrier()

    @pl.when(c == 0)
    def _():
        run(z0_hbm, y0_hbm, o0_hbm, False)
        run(z1_hbm, y1_hbm, o1_hbm, True)

    @pl.when(c == 1)
    def _():
        run(z2_hbm, y2_hbm, o2_hbm, False)
        run(z3_hbm, y3_hbm, o3_hbm, True)


def _edge_pass(src3, dst3, zs, ys, dinv):
    mesh = plsc.VectorSubcoreMesh(core_axis_name="c", subcore_axis_name="s")
    k = functools.partial(
        pl.kernel,
        mesh=mesh,
        compiler_params=pltpu.CompilerParams(use_tc_tiling_on_sc=False),
        out_type=[jax.ShapeDtypeStruct((N, C_OUT), _F32) for _ in range(4)],
        scratch_types=[
            pltpu.VMEM((NB_E, EB), jnp.int32),
            pltpu.VMEM((NB_E, EB), jnp.int32),
            pltpu.VMEM((EB, C_OUT), _F32),
            pltpu.VMEM((EB, C_OUT), _F32),
            pltpu.VMEM((EB, C_OUT), _F32),
            pltpu.VMEM((EB, C_OUT), _F32),
            pltpu.VMEM((EB, C_OUT), _F32),
            pltpu.VMEM((EB, 16), _F32),
            pltpu.VMEM_SHARED((N_PAD, C_OUT), _F32),
            pltpu.SemaphoreType.DMA,
            pltpu.SemaphoreType.DMA,
            pltpu.SemaphoreType.DMA,
            pltpu.SemaphoreType.DMA,
        ],
    )(_edge_body)
    return k(src3, dst3, *zs, *ys, dinv)


def kernel(x, edge_index, gamma, beta, W0, W1, b):
    xt = jnp.transpose(x, (2, 0, 1))

    src = edge_index[0]
    dst = edge_index[1]
    pad = E_PAD - E
    padidx = N + (jnp.arange(pad, dtype=jnp.int32) % 200)
    src3 = jnp.concatenate([src, padidx]).reshape(16, NB_E, EB)
    dst3 = jnp.concatenate([dst, padidx]).reshape(16, NB_E, EB)

    sums = _stats(xt)
    deg2d = _degree_inv(dst3)

    outs = _main(xt, sums, deg2d,
                 gamma.reshape(1, C_IN), beta.reshape(1, C_IN),
                 W0, W1, b.reshape(1, C_OUT))
    zs, ys, dinv = outs[0:4], outs[4:8], outs[8]

    o0, o1, o2, o3 = _edge_pass(src3, dst3, zs, ys, dinv)
    return jnp.stack([o0, o1, o2, o3], axis=2)


# speedup vs baseline: 20.5436x; 1.0075x over previous
import functools

import jax
import jax.numpy as jnp
from jax import lax
from jax.experimental import pallas as pl
from jax.experimental.pallas import tpu as pltpu
from jax.experimental.pallas import tpu_sc as plsc

N = 10000
E = 320000
C_IN = 128
C_OUT = 64
T = 4
THRESH = 0.5
TAU = 0.25
EPS = 1e-5

N_PAD = 10240
EB = 192
NB_E = 105
E_PAD = NB_E * EB * 16
WB = 128
NBLK = 1000
NT = float(N * T)

_F32 = jnp.float32


def _stats_body(xt_ref, sums_ref):
    blk = xt_ref[...]
    s1 = jnp.sum(blk, axis=(0, 1)).reshape(1, C_IN)
    s2 = jnp.sum(blk * blk, axis=(0, 1)).reshape(1, C_IN)
    upd = jnp.concatenate([s1, s2], axis=0)

    @pl.when(pl.program_id(0) == 0)
    def _():
        sums_ref[...] = upd

    @pl.when(pl.program_id(0) != 0)
    def _():
        sums_ref[...] = sums_ref[...] + upd


def _stats(xt):
    return pl.pallas_call(
        _stats_body,
        grid=(N // NBLK,),
        in_specs=[pl.BlockSpec((T, NBLK, C_IN), lambda g: (0, g, 0))],
        out_specs=pl.BlockSpec((2, C_IN), lambda g: (0, 0)),
        out_shape=jax.ShapeDtypeStruct((2, C_IN), _F32),
    )(xt)


def _deg_body(dst3_hbm, ones_hbm, zeros_hbm, deg_hbm, dst_v, ones_v, deg_sh,
              d0, d1, d2, d3):
    dsems = (d0, d1, d2, d3)
    c = lax.axis_index("c")
    s = lax.axis_index("s")

    pltpu.sync_copy(ones_hbm, ones_v)
    for k in range(5):
        pltpu.sync_copy(zeros_hbm, deg_sh.at[pl.ds(s * 640 + k * WB, WB)])
    plsc.subcore_barrier()

    pltpu.sync_copy(dst3_hbm.at[s], dst_v)

    def acc(p, carry):
        hs = [
            pltpu.async_copy(ones_v, deg_sh.at[dst_v.at[4 * p + q]],
                             dsems[q], add=True)
            for q in range(4)
        ]
        for h in hs:
            h.wait()
        return carry

    lax.fori_loop(0, NB_E // 4, acc, 0)
    for j in range((NB_E // 4) * 4, NB_E):
        pltpu.sync_copy(ones_v, deg_sh.at[dst_v.at[j]], add=True)
    plsc.subcore_barrier()

    base = (c * 16 + s) * 320
    pltpu.sync_copy(deg_sh.at[pl.ds(base, 320)],
                    deg_hbm.at[pl.ds(base, 320)])


def _degree_inv(dst3):
    mesh = plsc.VectorSubcoreMesh(core_axis_name="c", subcore_axis_name="s")
    k = functools.partial(
        pl.kernel,
        mesh=mesh,
        compiler_params=pltpu.CompilerParams(use_tc_tiling_on_sc=False),
        out_type=jax.ShapeDtypeStruct((N_PAD, 8), _F32),
        scratch_types=[
            pltpu.VMEM((NB_E, EB), jnp.int32),
            pltpu.VMEM((EB, 8), _F32),
            pltpu.VMEM_SHARED((N_PAD, 8), _F32),
            pltpu.SemaphoreType.DMA,
            pltpu.SemaphoreType.DMA,
            pltpu.SemaphoreType.DMA,
            pltpu.SemaphoreType.DMA,
        ],
    )(_deg_body)
    return k(dst3, jnp.ones((EB, 8), _F32), jnp.zeros((WB, 8), _F32))


def _main_body(xt_ref, sums_ref, deg_ref, gamma_ref, beta_ref, W0_ref,
               W1_ref, b_ref, z0_ref, z1_ref, z2_ref, z3_ref, y0_ref, y1_ref,
               y2_ref, y3_ref, dinv_ref):
    z_refs = (z0_ref, z1_ref, z2_ref, z3_ref)
    y_refs = (y0_ref, y1_ref, y2_ref, y3_ref)
    mean = sums_ref[0:1, :] / NT
    var = sums_ref[1:2, :] / NT - mean * mean
    inv = lax.rsqrt(var + EPS)
    scale = gamma_ref[...] * inv
    bias = beta_ref[...] - mean * scale
    deg = deg_ref[...][:, 0:1]
    dinv = jnp.where(deg > 0.0, lax.rsqrt(jnp.maximum(deg, 1e-30)), 0.0)
    dinv_ref[...] = jnp.broadcast_to(dinv, (NBLK, 16))
    W0 = W0_ref[...]
    W1 = W1_ref[...]
    bb = b_ref[...]

    mem = jnp.zeros((NBLK, C_IN), _F32)
    for t in range(T):
        h = xt_ref[t] * scale + bias
        mem = mem * TAU + h
        sp = (mem > THRESH).astype(_F32)
        mem = mem * (1.0 - sp)
        y_refs[t][...] = jnp.dot(sp, W0, preferred_element_type=_F32) + bb
        z_refs[t][...] = jnp.dot(sp, W1, preferred_element_type=_F32) * dinv


def _main(xt, sums, deg2d, gamma2d, beta2d, W0, W1, b2d):
    return pl.pallas_call(
        _main_body,
        grid=(N // NBLK,),
        in_specs=[
            pl.BlockSpec((T, NBLK, C_IN), lambda g: (0, g, 0)),
            pl.BlockSpec((2, C_IN), lambda g: (0, 0)),
            pl.BlockSpec((NBLK, 8), lambda g: (g, 0)),
            pl.BlockSpec((1, C_IN), lambda g: (0, 0)),
            pl.BlockSpec((1, C_IN), lambda g: (0, 0)),
            pl.BlockSpec((C_IN, C_OUT), lambda g: (0, 0)),
            pl.BlockSpec((C_IN, C_OUT), lambda g: (0, 0)),
            pl.BlockSpec((1, C_OUT), lambda g: (0, 0)),
        ],
        out_specs=(
            [pl.BlockSpec((NBLK, C_OUT), lambda g: (g, 0)) for _ in range(8)]
            + [pl.BlockSpec((NBLK, 16), lambda g: (g, 0))]
        ),
        out_shape=(
            [jax.ShapeDtypeStruct((N_PAD, C_OUT), _F32) for _ in range(4)]
            + [jax.ShapeDtypeStruct((N, C_OUT), _F32) for _ in range(4)]
            + [jax.ShapeDtypeStruct((N, 16), _F32)]
        ),
    )(xt, sums, deg2d, gamma2d, beta2d, W0, W1, b2d)


def _edge_body(src3_hbm, dst3_hbm, z0_hbm, z1_hbm, z2_hbm, z3_hbm, y0_hbm,
               y1_hbm, y2_hbm, y3_hbm, dinv_hbm, zeros_hbm, o0_hbm, o1_hbm,
               o2_hbm, o3_hbm, src_v, dst_v, rows_a, rows_b,
               acc_v, y_v, dinv_v, acc_sh, gsem_a, gsem_b, ssem_a, ssem_b):
    c = lax.axis_index("c")
    s = lax.axis_index("s")
    bufs = (rows_a, rows_b)
    sems = (gsem_a, gsem_b)
    ssems = (ssem_a, ssem_b)

    def zero_acc():
        for k in range(5):
            pltpu.sync_copy(zeros_hbm,
                            acc_sh.at[pl.ds(s * 640 + k * WB, WB)])

    zero_acc()
    plsc.subcore_barrier()

    pltpu.sync_copy(src3_hbm.at[s], src_v)
    pltpu.sync_copy(dst3_hbm.at[s], dst_v)

    def run(z_hbm, y_hbm, out_hbm, last):
        K = 2

        def group(p, carry):
            j0 = p * K
            hs = [
                pltpu.async_copy(z_hbm.at[src_v.at[j0 + q]], bufs[q],
                                 sems[q])
                for q in range(K)
            ]
            for q in range(K):
                hs[q].wait()
                pltpu.sync_copy(bufs[q], acc_sh.at[dst_v.at[j0 + q]],
                                add=True)
            return carry

        lax.fori_loop(0, NB_E // K, group, 0)

        for j in range((NB_E // K) * K, NB_E):
            pltpu.async_copy(z_hbm.at[src_v.at[j]], bufs[0],
                             sems[0]).wait()
            pltpu.sync_copy(bufs[0], acc_sh.at[dst_v.at[j]], add=True)
        plsc.subcore_barrier()

        def wb_chunk(r0, cn):
            pltpu.sync_copy(acc_sh.at[pl.ds(r0, cn)], acc_v.at[pl.ds(0, cn)])
            pltpu.sync_copy(y_hbm.at[pl.ds(r0, cn)], y_v.at[pl.ds(0, cn)])
            pltpu.sync_copy(dinv_hbm.at[pl.ds(r0, cn)],
                            dinv_v.at[pl.ds(0, cn)])

            def wb(i, carry):
                dv = dinv_v[i, :]
                for q in range(C_OUT // 16):
                    sl = pl.ds(q * 16, 16)
                    acc_v[i, sl] = y_v[i, sl] + acc_v[i, sl] * dv
                return carry

            lax.fori_loop(0, cn, wb, 0)
            pltpu.sync_copy(acc_v.at[pl.ds(0, cn)], out_hbm.at[pl.ds(r0, cn)])

        for k in range(5):
            cid = s + 16 * k

            @pl.when(cid < 78)
            def _():
                wb_chunk(cid * WB, WB)

            @pl.when(cid == 78)
            def _():
                wb_chunk(78 * WB, 16)

        if not last:
            plsc.subcore_barrier()
            zero_acc()
            plsc.subcore_barrier()

    @pl.when(c == 0)
    def _():
        run(z0_hbm, y0_hbm, o0_hbm, False)
        run(z1_hbm, y1_hbm, o1_hbm, True)

    @pl.when(c == 1)
    def _():
        run(z2_hbm, y2_hbm, o2_hbm, False)
        run(z3_hbm, y3_hbm, o3_hbm, True)


def _edge_pass(src3, dst3, zs, ys, dinv):
    mesh = plsc.VectorSubcoreMesh(core_axis_name="c", subcore_axis_name="s")
    k = functools.partial(
        pl.kernel,
        mesh=mesh,
        compiler_params=pltpu.CompilerParams(use_tc_tiling_on_sc=False),
        out_type=[jax.ShapeDtypeStruct((N, C_OUT), _F32) for _ in range(4)],
        scratch_types=[
            pltpu.VMEM((NB_E, EB), jnp.int32),
            pltpu.VMEM((NB_E, EB), jnp.int32),
            pltpu.VMEM((EB, C_OUT), _F32),
            pltpu.VMEM((EB, C_OUT), _F32),
            pltpu.VMEM((WB, C_OUT), _F32),
            pltpu.VMEM((WB, C_OUT), _F32),
            pltpu.VMEM((WB, 16), _F32),
            pltpu.VMEM_SHARED((N_PAD, C_OUT), _F32),
            pltpu.SemaphoreType.DMA,
            pltpu.SemaphoreType.DMA,
            pltpu.SemaphoreType.DMA,
            pltpu.SemaphoreType.DMA,
        ],
    )(_edge_body)
    return k(src3, dst3, *zs, *ys, dinv, jnp.zeros((WB, C_OUT), _F32))


def kernel(x, edge_index, gamma, beta, W0, W1, b):
    xt = jnp.transpose(x, (2, 0, 1))

    src = edge_index[0]
    dst = edge_index[1]
    pad = E_PAD - E
    padidx = N + (jnp.arange(pad, dtype=jnp.int32) % 200)
    src3 = jnp.concatenate([src, padidx]).reshape(16, NB_E, EB)
    dst3 = jnp.concatenate([dst, padidx]).reshape(16, NB_E, EB)

    sums = _stats(xt)
    deg2d = _degree_inv(dst3)

    outs = _main(xt, sums, deg2d,
                 gamma.reshape(1, C_IN), beta.reshape(1, C_IN),
                 W0, W1, b.reshape(1, C_OUT))
    zs, ys, dinv = outs[0:4], outs[4:8], outs[8]

    o0, o1, o2, o3 = _edge_pass(src3, dst3, zs, ys, dinv)
    return jnp.stack([o0, o1, o2, o3], axis=2)
